# Initial kernel scaffold; baseline (speedup 1.0000x reference)
import jax
import jax.numpy as jnp
from jax.experimental import pallas as pl


def kernel(pts, params):
    raise NotImplementedError("write your pallas kernel here")


# speedup vs baseline: 1.0002x; 1.0002x over previous
import jax, jax.numpy as jnp
from jax.experimental import pallas as pl

NUM_CLASSES = 13
NSAMPLE = 32


def _cdist(a, b):
    a2 = jnp.sum(a * a, axis=-1)
    b2 = jnp.sum(b * b, axis=-1)
    d2 = a2[:, :, None] + b2[:, None, :] - 2.0 * jnp.einsum('bnd,bmd->bnm', a, b)
    return jnp.sqrt(jnp.maximum(d2, 0.0))


def _bn(x, g, beta):
    axes = (0,) + tuple(range(2, x.ndim))
    m = jnp.mean(x, axis=axes, keepdims=True)
    v = jnp.var(x, axis=axes, keepdims=True)
    shape = (1, -1) + (1,) * (x.ndim - 2)
    return g.reshape(shape) * (x - m) / jnp.sqrt(v + 1e-5) + beta.reshape(shape)


def _conv1(x, W, b):
    return jnp.einsum('oc,bcn->bon', W, x) + b[None, :, None]


def _conv2(x, W, b):
    return jnp.einsum('oc,bcmk->bomk', W, x) + b[None, :, None, None]


def _mlp(x, layers, conv):
    for (W, b, g, bt) in layers:
        x = jax.nn.relu(_bn(conv(x, W, b), g, bt))
    return x


def _sa(xyz, feat_in, layers, nsample):
    B, N, _ = xyz.shape
    M = max(1, N // 4)
    idx_center = jnp.linspace(0.0, N - 1, M).astype(jnp.int32)
    centers = xyz[:, idx_center, :]
    d = _cdist(centers, xyz)
    k = min(nsample, N)
    negd, knn_idx = jax.lax.top_k(-d, k)
    gather = jax.vmap(lambda arr, i: arr[i])
    neigh_xyz = gather(xyz, knn_idx)
    local = neigh_xyz - centers[:, :, None, :]
    if feat_in is None:
        cat = local
    else:
        fT = jnp.transpose(feat_in, (0, 2, 1))
        neigh_f = gather(fT, knn_idx)
        cat = jnp.concatenate([local, neigh_f], axis=-1)
    cat = jnp.transpose(cat, (0, 3, 1, 2))
    f = _mlp(cat, layers, _conv2)
    f = jnp.max(f, axis=-1)
    return centers, f


def _fp(xyz_low, xyz_high, feat_low, feat_high, layers):
    Nh = xyz_high.shape[1]
    d = _cdist(xyz_low, xyz_high)
    k = min(3, Nh)
    negd, idx = jax.lax.top_k(-d, k)
    dist = jnp.maximum(-negd, 1e-8)
    w = 1.0 / dist
    w = w / jnp.sum(w, axis=-1, keepdims=True)
    fT = jnp.transpose(feat_high, (0, 2, 1))
    neigh = jax.vmap(lambda arr, i: arr[i])(fT, idx)
    f_interp = jnp.sum(w[..., None] * neigh, axis=2)
    f_interp = jnp.transpose(f_interp, (0, 2, 1))
    if feat_low is not None:
        f_interp = jnp.concatenate([f_interp, feat_low], axis=1)
    return _mlp(f_interp, layers, _conv1)


def _copy_kernel(x_ref, o_ref):
    o_ref[...] = x_ref[...]


def kernel(pts, params):
    xyz = pts[:, :, :3]
    feats0 = jnp.transpose(pts[:, :, 3:], (0, 2, 1)) if pts.shape[2] > 3 else None
    xyz1, f1 = _sa(xyz, feats0, params['sa1'], NSAMPLE)
    xyz2, f2 = _sa(xyz1, f1, params['sa2'], NSAMPLE)
    fu1 = _fp(xyz1, xyz2, f1, f2, params['fp1'])
    fu0 = _fp(xyz, xyz1, None, fu1, params['fp2'])
    W1, b1, g1, bt1, W2, b2 = params['head']
    h = jax.nn.relu(_bn(_conv1(fu0, W1, b1), g1, bt1))
    out = _conv1(h, W2, b2)
    out = jnp.transpose(out, (0, 2, 1))
    out = pl.pallas_call(
        _copy_kernel,
        out_shape=jax.ShapeDtypeStruct(out.shape, out.dtype),
    )(out)
    return out


# speedup vs baseline: 2.5804x; 2.5798x over previous
import functools

import jax
import jax.numpy as jnp
from jax.experimental import pallas as pl

NUM_CLASSES = 13
NSAMPLE = 32
_EPS = 1e-5
_INF = 3e38


def _knn_body(c_ref, pt_ref, idx_ref, k, n, want_w, w_ref=None):
    c = c_ref[...].reshape(c_ref.shape[1], 8)
    p = pt_ref[...].reshape(8, n)
    c2 = jnp.sum(c * c, axis=1, keepdims=True)
    p2 = jnp.sum(p * p, axis=0, keepdims=True)
    cp = jnp.dot(c.astype(jnp.bfloat16), p.astype(jnp.bfloat16),
                 preferred_element_type=jnp.float32)
    d = jnp.sqrt(jnp.maximum(c2 + p2 - 2.0 * cp, 0.0))
    iota = jax.lax.broadcasted_iota(jnp.int32, d.shape, 1)
    nbig = jnp.int32(n)
    idxs = []
    vals = []
    for _ in range(k):
        m = jnp.min(d, axis=1, keepdims=True)
        cand = jnp.where(d == m, iota, nbig)
        amin = jnp.min(cand, axis=1, keepdims=True)
        idxs.append(amin)
        vals.append(m)
        d = jnp.where(cand == amin, _INF, d)
    idx_ref[...] = jnp.concatenate(idxs, axis=1)[None]
    if want_w:
        w_ref[...] = jnp.concatenate(vals, axis=1)[None]


def _inv_dist_weights(d3):
    dist = jnp.maximum(d3, 1e-8)
    w = 1.0 / dist
    return w / jnp.sum(w, axis=-1, keepdims=True)


def _knn(centers8, ptsT8, k, tm, want_w):
    b, m, _ = centers8.shape
    n = ptsT8.shape[2]
    grid = (b, m // tm)
    out_shape = [jax.ShapeDtypeStruct((b, m, k), jnp.int32)]
    out_specs = [pl.BlockSpec((1, tm, k), lambda bi, mi: (bi, mi, 0))]
    if want_w:
        out_shape.append(jax.ShapeDtypeStruct((b, m, k), jnp.float32))
        out_specs.append(pl.BlockSpec((1, tm, k), lambda bi, mi: (bi, mi, 0)))
    body = functools.partial(_knn_body, k=k, n=n, want_w=want_w)
    if want_w:
        fn = lambda c_ref, p_ref, i_ref, w_ref: body(c_ref, p_ref, i_ref, w_ref=w_ref)
    else:
        fn = lambda c_ref, p_ref, i_ref: body(c_ref, p_ref, i_ref)
    res = pl.pallas_call(
        fn,
        grid=grid,
        in_specs=[
            pl.BlockSpec((1, tm, 8), lambda bi, mi: (bi, mi, 0)),
            pl.BlockSpec((1, 8, n), lambda bi, mi: (bi, 0, 0)),
        ],
        out_specs=out_specs,
        out_shape=out_shape,
    )(centers8, ptsT8)
    return res


def _gather_cat_body(idx_ref, tab_ref, cen_ref, out_ref, n, m, tm):
    idx = idx_ref[...].reshape(tm, 1)
    tab = tab_ref[...].reshape(n, tab_ref.shape[2])
    cen = cen_ref[...].reshape(tm, cen_ref.shape[2])
    iota = jax.lax.broadcasted_iota(jnp.int32, (tm, n), 1)
    onehot = (iota == idx).astype(jnp.float32)
    rows = jnp.dot(onehot, tab, preferred_element_type=jnp.float32,
                   precision=jax.lax.Precision.HIGHEST)
    out_ref[...] = (rows - cen)[None]


def _gather_cat(idx_col, table, centers_pad, tm):
    b, r, _ = idx_col.shape
    _, n, c = table.shape
    mctr = centers_pad.shape[1]
    grid = (b, r // tm)
    body = functools.partial(_gather_cat_body, n=n, m=mctr, tm=tm)
    nmb = mctr // tm
    return pl.pallas_call(
        body,
        grid=grid,
        in_specs=[
            pl.BlockSpec((1, tm, 1), lambda bi, ri: (bi, ri, 0)),
            pl.BlockSpec((1, n, c), lambda bi, ri: (bi, 0, 0)),
            pl.BlockSpec((1, tm, c), lambda bi, ri, _nmb=nmb: (bi, ri % _nmb, 0)),
        ],
        out_specs=pl.BlockSpec((1, tm, c), lambda bi, ri: (bi, ri, 0)),
        out_shape=jax.ShapeDtypeStruct((b, r, c), jnp.float32),
    )(idx_col, table, centers_pad)


def _bn_vectors(stats, g, bt, n):
    mean = stats[0:1, :] / n
    var = stats[1:2, :] / n - mean * mean
    return mean, jnp.sqrt(var + _EPS), g, bt


def _bn_relu(x, m_ref, s_ref, g_ref, bt_ref):
    return jnp.maximum(g_ref[...] * (x - m_ref[...]) / s_ref[...] + bt_ref[...], 0.0)


def _conv_body(*refs, prologue, emit_stats):
    it = iter(refs)
    x_ref, wt_ref, b_ref = next(it), next(it), next(it)
    if prologue:
        m_ref, s_ref, g_ref, bt_ref = next(it), next(it), next(it), next(it)
    out_ref = next(it)
    if emit_stats:
        stats_ref = next(it)
    x = x_ref[...]
    if prologue:
        x = _bn_relu(x, m_ref, s_ref, g_ref, bt_ref)
    y = jnp.dot(x.astype(jnp.bfloat16), wt_ref[...].astype(jnp.bfloat16),
                preferred_element_type=jnp.float32) + b_ref[...]
    out_ref[...] = y
    if emit_stats:
        step = pl.program_id(0)

        @pl.when(step == 0)
        def _():
            stats_ref[...] = jnp.zeros_like(stats_ref)

        s1 = jnp.sum(y, axis=0, keepdims=True)
        s2 = jnp.sum(y * y, axis=0, keepdims=True)
        part = jnp.concatenate([s1, s2], axis=0)
        s = stats_ref[0:2, :]
        comp = stats_ref[2:4, :]
        yv = part - comp
        t = s + yv
        comp_new = (t - s) - yv
        stats_ref[0:2, :] = t
        stats_ref[2:4, :] = comp_new


def _conv_pass(x, wt, bias, prev=None, emit_stats=True, tr=256):
    r, cin = x.shape
    cout = wt.shape[1]
    grid = (r // tr,)
    in_specs = [
        pl.BlockSpec((tr, cin), lambda i: (i, 0)),
        pl.BlockSpec((cin, cout), lambda i: (0, 0)),
        pl.BlockSpec((1, cout), lambda i: (0, 0)),
    ]
    args = [x, wt, bias]
    if prev is not None:
        in_specs += [pl.BlockSpec((1, cin), lambda i: (0, 0))] * 4
        args += list(prev)
    out_shape = [jax.ShapeDtypeStruct((r, cout), jnp.float32)]
    out_specs = [pl.BlockSpec((tr, cout), lambda i: (i, 0))]
    if emit_stats:
        out_shape.append(jax.ShapeDtypeStruct((4, cout), jnp.float32))
        out_specs.append(pl.BlockSpec((4, cout), lambda i: (0, 0)))
    body = functools.partial(
        _conv_body, prologue=prev is not None, emit_stats=emit_stats)
    return pl.pallas_call(
        body,
        grid=grid,
        in_specs=in_specs,
        out_specs=out_specs,
        out_shape=out_shape,
    )(*args)


def _maxpool_body(x_ref, m_ref, s_ref, g_ref, bt_ref, out_ref):
    y = _bn_relu(x_ref[...], m_ref, s_ref, g_ref, bt_ref)
    k = pl.program_id(2)

    @pl.when(k == 0)
    def _():
        out_ref[...] = y

    @pl.when(k != 0)
    def _():
        out_ref[...] = jnp.maximum(out_ref[...], y)


def _maxpool(x, bnv, b, kk, m, tm=128):
    c = x.shape[2]
    grid = (b, m // tm, kk)
    nmb = m // tm
    return pl.pallas_call(
        _maxpool_body,
        grid=grid,
        in_specs=[
            pl.BlockSpec((1, tm, c), lambda bi, mi, ki, _nmb=nmb: (bi, ki * _nmb + mi, 0)),
        ] + [pl.BlockSpec((1, c), lambda bi, mi, ki: (0, 0))] * 4,
        out_specs=pl.BlockSpec((1, tm, c), lambda bi, mi, ki: (bi, mi, 0)),
        out_shape=jax.ShapeDtypeStruct((b, m, c), jnp.float32),
    )(x, *bnv)


def _interp_body(idx_ref, w_ref, tab_ref, out_ref, nh, tm, prologue, *rest):
    tab = tab_ref[...].reshape(nh, tab_ref.shape[2])
    if prologue:
        m_ref, s_ref, g_ref, bt_ref = rest
        tab = _bn_relu(tab, m_ref, s_ref, g_ref, bt_ref)
    idx = idx_ref[...].reshape(tm, 3)
    w = w_ref[...].reshape(tm, 3)
    iota = jax.lax.broadcasted_iota(jnp.int32, (tm, nh), 1)
    neigh = []
    for j in range(3):
        sel = (iota == idx[:, j:j + 1]).astype(jnp.float32)
        neigh.append(jnp.dot(sel, tab, preferred_element_type=jnp.float32,
                             precision=jax.lax.Precision.HIGHEST))
    acc = w[:, 0:1] * neigh[0] + w[:, 1:2] * neigh[1] + w[:, 2:3] * neigh[2]
    out_ref[...] = acc[None]


def _interp(idx3, w3, table, prev=None, tm=256):
    b, r, _ = idx3.shape
    _, nh, c = table.shape
    grid = (b, r // tm)
    in_specs = [
        pl.BlockSpec((1, tm, 3), lambda bi, ri: (bi, ri, 0)),
        pl.BlockSpec((1, tm, 3), lambda bi, ri: (bi, ri, 0)),
        pl.BlockSpec((1, nh, c), lambda bi, ri: (bi, 0, 0)),
    ]
    args = [idx3, w3, table]
    prologue = prev is not None
    if prologue:
        in_specs += [pl.BlockSpec((1, c), lambda bi, ri: (0, 0))] * 4
        args += list(prev)

    def body(i_ref, ww_ref, t_ref, *r2):
        if prologue:
            m_ref, s_ref, g_ref, bt_ref, out_ref = r2
            _interp_body(i_ref, ww_ref, t_ref, out_ref, nh, tm, True,
                         m_ref, s_ref, g_ref, bt_ref)
        else:
            (out_ref,) = r2
            _interp_body(i_ref, ww_ref, t_ref, out_ref, nh, tm, False)

    return pl.pallas_call(
        body,
        grid=grid,
        in_specs=in_specs,
        out_specs=pl.BlockSpec((1, tm, c), lambda bi, ri: (bi, ri, 0)),
        out_shape=jax.ShapeDtypeStruct((b, r, c), jnp.float32),
    )(*args)


def _pad_last(x, to):
    return jnp.pad(x, [(0, 0)] * (x.ndim - 1) + [(0, to - x.shape[-1])])


def _wt(w, cin_pad):
    wt = jnp.transpose(w, (1, 0))
    return jnp.pad(wt, [(0, cin_pad - wt.shape[0]), (0, 0)])


def _idx_col(knn_idx):
    b, m, k = knn_idx.shape
    return jnp.transpose(knn_idx, (0, 2, 1)).reshape(b, k * m, 1)


def _sa_stage(xyz, table, centers, layers, b, n, m, k, tm_knn):
    cpad = table.shape[2]
    centers8 = _pad_last(centers, 8)
    ptsT8 = jnp.transpose(_pad_last(xyz, 8), (0, 2, 1))
    (knn_idx,) = _knn(centers8, ptsT8, k, tm_knn, want_w=False)
    cat = _gather_cat(_idx_col(knn_idx), table, _pad_last(centers, cpad), 128)
    rows = cat.reshape(b * k * m, cpad)
    nrows = float(b * k * m)
    (w1, b1, g1, bt1), (w2, b2, g2, bt2), (w3, b3, g3, bt3) = layers
    y1, st1 = _conv_pass(rows, _wt(w1, cpad), b1[None])
    y2, st2 = _conv_pass(y1, _wt(w2, w1.shape[0]), b2[None],
                         prev=_bn_vectors(st1, g1[None], bt1[None], nrows))
    y3, st3 = _conv_pass(y2, _wt(w3, w2.shape[0]), b3[None],
                         prev=_bn_vectors(st2, g2[None], bt2[None], nrows))
    cout = w3.shape[0]
    f = _maxpool(y3.reshape(b, k * m, cout),
                 _bn_vectors(st3, g3[None], bt3[None], nrows), b, k, m)
    return f


def kernel(pts, params):
    b, n, _ = pts.shape
    xyz = pts[:, :, :3]

    m1 = n // 4
    idx_c1 = jnp.linspace(0.0, n - 1, m1).astype(jnp.int32)
    centers1 = xyz[:, idx_c1, :]
    table1 = _pad_last(pts, 8)
    f1 = _sa_stage(xyz, table1, centers1, params['sa1'], b, n, m1, NSAMPLE, 128)

    m2 = m1 // 4
    idx_c2 = jnp.linspace(0.0, m1 - 1, m2).astype(jnp.int32)
    centers2 = centers1[:, idx_c2, :]
    c1f = f1.shape[2]
    table2 = _pad_last(jnp.concatenate([centers1, f1], axis=2), 384)
    f2 = _sa_stage(centers1, table2, centers2, params['sa2'], b, m1, m2,
                   NSAMPLE, 128)

    c1_8 = _pad_last(centers1, 8)
    c2T8 = jnp.transpose(_pad_last(centers2, 8), (0, 2, 1))
    idx3a, d3a = _knn(c1_8, c2T8, 3, 256, want_w=True)
    fi1 = _interp(idx3a, _inv_dist_weights(d3a), f2)
    rows_fp1 = jnp.concatenate([fi1, f1], axis=2).reshape(b * m1, -1)
    n_fp1 = float(b * m1)
    (wa, ba, ga, bta), (wb, bb, gb, btb) = params['fp1']
    ya, sta = _conv_pass(rows_fp1, _wt(wa, rows_fp1.shape[1]), ba[None])
    yb, stb = _conv_pass(ya, _wt(wb, wa.shape[0]), bb[None],
                         prev=_bn_vectors(sta, ga[None], bta[None], n_fp1))

    xyz8 = _pad_last(xyz, 8)
    c1T8 = jnp.transpose(c1_8, (0, 2, 1))
    idx3b, d3b = _knn(xyz8, c1T8, 3, 256, want_w=True)
    fu1_tab = yb.reshape(b, m1, wb.shape[0])
    fi2 = _interp(idx3b, _inv_dist_weights(d3b), fu1_tab,
                  prev=_bn_vectors(stb, gb[None], btb[None], n_fp1))
    rows_fp2 = fi2.reshape(b * n, -1)
    n_fp2 = float(b * n)
    (wc, bc, gc, btc), (wd, bd, gd, btd) = params['fp2']
    yc, stc = _conv_pass(rows_fp2, _wt(wc, wc.shape[1]), bc[None])
    yd, std = _conv_pass(yc, _wt(wd, wc.shape[0]), bd[None],
                         prev=_bn_vectors(stc, gc[None], btc[None], n_fp2))

    wh1, bh1, gh1, bth1, wh2, bh2 = params['head']
    yh, sth = _conv_pass(yd, _wt(wh1, wd.shape[0]), bh1[None],
                         prev=_bn_vectors(std, gd[None], btd[None], n_fp2))
    wt2 = jnp.pad(jnp.transpose(wh2, (1, 0)), [(0, 0), (0, 128 - NUM_CLASSES)])
    b2p = jnp.pad(bh2, [(0, 128 - NUM_CLASSES)])
    (out,) = _conv_pass(yh, wt2, b2p[None],
                        prev=_bn_vectors(sth, gh1[None], bth1[None], n_fp2),
                        emit_stats=False)
    return out.reshape(b, n, 128)[:, :, :NUM_CLASSES]
